# baseline (speedup 1.0000x reference)
import functools

import jax
import jax.numpy as jnp
from jax import lax
from jax.experimental import pallas as pl
from jax.experimental.pallas import tpu as pltpu
from jax.experimental.pallas import tpu_sc as plsc

_S = 64.0
_M = 0.4

_TC_BLK = 819200


def _scale_body(x_ref, o_ref):
    o_ref[...] = x_ref[...] * jnp.float32(_S)


def _tc_scale(flat):
    n = flat.shape[0]
    blk = _TC_BLK if n % _TC_BLK == 0 else n
    return pl.pallas_call(
        _scale_body,
        out_shape=jax.ShapeDtypeStruct((n,), jnp.float32),
        grid=(n // blk,),
        in_specs=[pl.BlockSpec((blk,), lambda i: (i,))],
        out_specs=pl.BlockSpec((blk,), lambda i: (i,)),
    )(flat)


@functools.cache
def _sc_fixup(rows, vocab):
    info = plsc.get_sparse_core_info()
    nc, lanes = info.num_cores, info.num_lanes
    nw = nc * info.num_subcores
    per_w = rows // nw
    mesh = plsc.VectorSubcoreMesh(core_axis_name="c", subcore_axis_name="s")

    @functools.partial(
        pl.kernel,
        out_type=(),
        mesh=mesh,
        scratch_types=[
            pltpu.VMEM((per_w,), jnp.int32),
            pltpu.VMEM((per_w,), jnp.int32),
            pltpu.VMEM((per_w,), jnp.float32),
            pltpu.SemaphoreType.DMA,
        ],
    )
    def fixup(lab_hbm, data_hbm, lab_v, idx_v, val_v, sem):
        wid = lax.axis_index("s") * nc + lax.axis_index("c")
        base = wid * per_w
        pltpu.sync_copy(lab_hbm.at[pl.ds(base, per_w)], lab_v)
        for k in range(per_w // lanes):
            lab = lab_v[pl.ds(k * lanes, lanes)]
            row = base + k * lanes + lax.iota(jnp.int32, lanes)
            idx_v[pl.ds(k * lanes, lanes)] = row * vocab + jnp.maximum(lab, 0)
        pltpu.async_copy(data_hbm.at[idx_v], val_v, sem).wait()
        for k in range(per_w // lanes):
            lab = lab_v[pl.ds(k * lanes, lanes)]
            val = val_v[pl.ds(k * lanes, lanes)]
            margin = jnp.where(lab >= 0, jnp.float32(_S * _M), jnp.float32(0.0))
            val_v[pl.ds(k * lanes, lanes)] = val - margin
        pltpu.async_copy(val_v, data_hbm.at[idx_v], sem).wait()

    return fixup


def kernel(logits, labels):
    rows, vocab = logits.shape
    scaled = _tc_scale(logits.reshape(-1))
    out_ref = jax.new_ref(scaled)
    _sc_fixup(rows, vocab)(labels.astype(jnp.int32), out_ref)
    return jax.freeze(out_ref).reshape(rows, vocab)
